# baseline (speedup 1.0000x reference)
import functools

import jax
import jax.numpy as jnp
from jax import lax
from jax.experimental import pallas as pl
from jax.experimental.pallas import tpu as pltpu
from jax.experimental.pallas import tpu_sc as plsc

NC = 2
NS = 16
NW = NC * NS
CE = 128
L = 16


def _mesh():
  return plsc.VectorSubcoreMesh(core_axis_name="c", subcore_axis_name="s")


def _deg_call(npad, chunks):
  rps = npad // NS

  @functools.partial(
      pl.kernel,
      out_type=jax.ShapeDtypeStruct((NC, npad, 16), jnp.float32),
      mesh=_mesh(),
      compiler_params=pltpu.CompilerParams(use_tc_tiling_on_sc=False),
      scratch_types=[
          pltpu.VMEM((chunks, CE), jnp.int32),
          pltpu.VMEM((CE, 16), jnp.float32),
          pltpu.VMEM_SHARED((npad, 16), jnp.float32),
      ],
  )
  def k(zd_hbm, one_hbm, dst_hbm, out_hbm, didx, ones, accd):
    c = lax.axis_index("c")
    s = lax.axis_index("s")
    wid = s * NC + c
    pltpu.sync_copy(zd_hbm.at[pl.ds(s * rps, rps)], accd.at[pl.ds(s * rps, rps)])
    pltpu.sync_copy(dst_hbm.at[wid], didx)
    pltpu.sync_copy(one_hbm, ones)
    plsc.subcore_barrier()

    def body(i, carry):
      pltpu.sync_copy(ones, accd.at[didx.at[i]], add=True)
      return carry

    lax.fori_loop(0, chunks, body, 0)
    plsc.subcore_barrier()
    pltpu.sync_copy(accd.at[pl.ds(s * rps, rps)],
                    out_hbm.at[c, pl.ds(s * rps, rps)])

  return k


def _seg_sum_call(npad, chunks):
  rps = npad // NS

  @functools.partial(
      pl.kernel,
      out_type=jax.ShapeDtypeStruct((NC, npad, 128), jnp.float32),
      mesh=_mesh(),
      scratch_types=[
          pltpu.VMEM((chunks, CE), jnp.int32),
          pltpu.VMEM((chunks, CE), jnp.int32),
          pltpu.VMEM((CE, 128), jnp.float32),
          pltpu.VMEM_SHARED((npad, 128), jnp.float32),
          pltpu.SemaphoreType.DMA,
      ],
  )
  def k(y_hbm, z_hbm, src_hbm, dst_hbm, out_hbm, sidx, didx, rows, acc, sem):
    c = lax.axis_index("c")
    s = lax.axis_index("s")
    wid = s * NC + c
    pltpu.sync_copy(z_hbm.at[pl.ds(s * rps, rps)], acc.at[pl.ds(s * rps, rps)])
    pltpu.sync_copy(src_hbm.at[wid], sidx)
    pltpu.sync_copy(dst_hbm.at[wid], didx)
    plsc.subcore_barrier()

    def body(i, carry):
      pltpu.async_copy(y_hbm.at[sidx.at[i]], rows, sem).wait()
      pltpu.sync_copy(rows, acc.at[didx.at[i]], add=True)
      return carry

    lax.fori_loop(0, chunks, body, 0)
    plsc.subcore_barrier()
    pltpu.sync_copy(acc.at[pl.ds(s * rps, rps)],
                    out_hbm.at[c, pl.ds(s * rps, rps)])

  return k


def _gat_den_call(npad, chunks):
  rps = npad // NS

  @functools.partial(
      pl.kernel,
      out_type=jax.ShapeDtypeStruct((NC, npad, 16), jnp.float32),
      mesh=_mesh(),
      compiler_params=pltpu.CompilerParams(use_tc_tiling_on_sc=False),
      scratch_types=[
          pltpu.VMEM((2, CE), jnp.int32),
          pltpu.VMEM((2, CE), jnp.int32),
          pltpu.VMEM((16,), jnp.float32),
          pltpu.VMEM((2, CE, 16), jnp.float32),
          pltpu.VMEM((2, CE, 16), jnp.float32),
          pltpu.VMEM((2, CE, 16), jnp.float32),
          pltpu.VMEM((CE, 16), jnp.float32),
          pltpu.VMEM_SHARED((npad, 16), jnp.float32),
          pltpu.SemaphoreType.DMA((2,)),
          pltpu.SemaphoreType.DMA((2,)),
      ],
  )
  def k(zd_hbm, src_hbm, dst_hbm, ae_hbm, asn_hbm, adn_hbm, avec_hbm,
        outd_hbm, sidx, didx, avecv, asr, adr, aer, exrows, accd, ixs, gds):
    c = lax.axis_index("c")
    s = lax.axis_index("s")
    wid = s * NC + c
    pltpu.sync_copy(zd_hbm.at[pl.ds(s * rps, rps)], accd.at[pl.ds(s * rps, rps)])
    pltpu.sync_copy(avec_hbm, avecv)
    plsc.subcore_barrier()
    A = avecv[...]

    def issue_idx(i, slot):
      pltpu.async_copy(src_hbm.at[wid, i], sidx.at[slot], ixs.at[slot])
      pltpu.async_copy(dst_hbm.at[wid, i], didx.at[slot], ixs.at[slot])

    def wait_idx(slot):
      pltpu.make_async_copy(src_hbm.at[0, 0], sidx.at[slot],
                            ixs.at[slot]).wait()
      pltpu.make_async_copy(src_hbm.at[0, 0], didx.at[slot],
                            ixs.at[slot]).wait()

    def issue_data(i, slot):
      pltpu.async_copy(asn_hbm.at[sidx.at[slot]], asr.at[slot], gds.at[slot])
      pltpu.async_copy(adn_hbm.at[didx.at[slot]], adr.at[slot], gds.at[slot])
      pltpu.async_copy(ae_hbm.at[wid, i], aer.at[slot], gds.at[slot])

    def wait_data(slot):
      for buf in (asr, adr, aer):
        pltpu.make_async_copy(asn_hbm.at[pl.ds(0, CE)], buf.at[slot],
                              gds.at[slot]).wait()

    issue_idx(0, 0)
    issue_idx(1, 1)
    wait_idx(0)
    issue_data(0, 0)

    def group(gi, carry):
      base = gi * 2
      for b in range(2):
        i = base + b
        o = 1 - b

        @pl.when(i + 1 < chunks)
        def _(i=i, o=o):
          wait_idx(o)
          issue_data(i + 1, o)

        wait_data(b)

        def edge_body(j, cc):
          al = asr[b, j, :] + adr[b, j, :] + aer[b, j, :]
          al = jnp.maximum(al, 0.2 * al)
          exrows[j, :] = jnp.exp(al - A)
          return cc

        lax.fori_loop(0, CE, edge_body, 0)
        pltpu.sync_copy(exrows, accd.at[didx.at[b]], add=True)

        @pl.when(i + 2 < chunks)
        def _(i=i, b=b):
          issue_idx(i + 2, b)

      return carry

    lax.fori_loop(0, chunks // 2, group, 0)
    plsc.subcore_barrier()
    pltpu.sync_copy(accd.at[pl.ds(s * rps, rps)],
                    outd_hbm.at[c, pl.ds(s * rps, rps)])

  return k


def _gat_num_call(npad, chunks):
  rps = npad // NS

  @functools.partial(
      pl.kernel,
      out_type=jax.ShapeDtypeStruct((NC, npad, 128), jnp.float32),
      mesh=_mesh(),
      compiler_params=pltpu.CompilerParams(use_tc_tiling_on_sc=False),
      scratch_types=[
          pltpu.VMEM((2, CE), jnp.int32),
          pltpu.VMEM((2, CE), jnp.int32),
          pltpu.VMEM((16,), jnp.float32),
          pltpu.VMEM((2, CE, 128), jnp.float32),
          pltpu.VMEM((2, CE, 16), jnp.float32),
          pltpu.VMEM((2, CE, 16), jnp.float32),
          pltpu.VMEM((2, CE, 16), jnp.float32),
          pltpu.VMEM_SHARED((npad, 128), jnp.float32),
          pltpu.SemaphoreType.DMA((2,)),
          pltpu.SemaphoreType.DMA((2,)),
      ],
  )
  def k(xs_hbm, zn_hbm, src_hbm, dst_hbm, ae_hbm, asn_hbm, adn_hbm,
        avec_hbm, outn_hbm, sidx, didx, avecv, rows, asr, adr, aer, accn,
        ixs, gds):
    c = lax.axis_index("c")
    s = lax.axis_index("s")
    wid = s * NC + c
    pltpu.sync_copy(zn_hbm.at[pl.ds(s * rps, rps)], accn.at[pl.ds(s * rps, rps)])
    pltpu.sync_copy(avec_hbm, avecv)
    plsc.subcore_barrier()
    A = avecv[...]

    def issue_idx(i, slot):
      pltpu.async_copy(src_hbm.at[wid, i], sidx.at[slot], ixs.at[slot])
      pltpu.async_copy(dst_hbm.at[wid, i], didx.at[slot], ixs.at[slot])

    def wait_idx(slot):
      pltpu.make_async_copy(src_hbm.at[0, 0], sidx.at[slot],
                            ixs.at[slot]).wait()
      pltpu.make_async_copy(src_hbm.at[0, 0], didx.at[slot],
                            ixs.at[slot]).wait()

    def issue_data(i, slot):
      pltpu.async_copy(xs_hbm.at[sidx.at[slot]], rows.at[slot], gds.at[slot])
      pltpu.async_copy(asn_hbm.at[sidx.at[slot]], asr.at[slot], gds.at[slot])
      pltpu.async_copy(adn_hbm.at[didx.at[slot]], adr.at[slot], gds.at[slot])
      pltpu.async_copy(ae_hbm.at[wid, i], aer.at[slot], gds.at[slot])

    def wait_data(slot):
      pltpu.make_async_copy(xs_hbm.at[pl.ds(0, CE)], rows.at[slot],
                            gds.at[slot]).wait()
      for buf in (asr, adr, aer):
        pltpu.make_async_copy(asn_hbm.at[pl.ds(0, CE)], buf.at[slot],
                              gds.at[slot]).wait()

    issue_idx(0, 0)
    issue_idx(1, 1)
    wait_idx(0)
    issue_data(0, 0)

    def group(gi, carry):
      base = gi * 2
      for b in range(2):
        i = base + b
        o = 1 - b

        @pl.when(i + 1 < chunks)
        def _(i=i, o=o):
          wait_idx(o)
          issue_data(i + 1, o)

        wait_data(b)

        def edge_body(j, cc):
          al = asr[b, j, :] + adr[b, j, :] + aer[b, j, :]
          al = jnp.maximum(al, 0.2 * al)
          exr = jnp.exp(al - A)
          for r in range(128 // L):
            rsl = pl.ds(r * L, L)
            rows[b, j, rsl] = rows[b, j, rsl] * exr
          return cc

        lax.fori_loop(0, CE, edge_body, 0)
        pltpu.sync_copy(rows.at[b], accn.at[didx.at[b]], add=True)

        @pl.when(i + 2 < chunks)
        def _(i=i, b=b):
          issue_idx(i + 2, b)

      return carry

    lax.fori_loop(0, chunks // 2, group, 0)
    plsc.subcore_barrier()
    pltpu.sync_copy(accn.at[pl.ds(s * rps, rps)],
                    outn_hbm.at[c, pl.ds(s * rps, rps)])

  return k


BN = 128


def _tc_pre_body(x_ref, w_ref, degp_ref, y_ref, dinv_ref):
  deg = degp_ref[0, :, 0:1] + degp_ref[1, :, 0:1] + 1.0
  dinv = lax.rsqrt(jnp.maximum(deg, 1.0))
  y_ref[...] = jnp.dot(x_ref[...], w_ref[...],
                       preferred_element_type=jnp.float32) * dinv
  dinv_ref[...] = dinv


def _tc_pre(npad):
  return pl.pallas_call(
      _tc_pre_body,
      grid=(npad // BN,),
      in_specs=[
          pl.BlockSpec((BN, 128), lambda i: (i, 0)),
          pl.BlockSpec((128, 128), lambda i: (0, 0)),
          pl.BlockSpec((2, BN, 16), lambda i: (0, i, 0)),
      ],
      out_specs=[
          pl.BlockSpec((BN, 128), lambda i: (i, 0)),
          pl.BlockSpec((BN, 1), lambda i: (i, 0)),
      ],
      out_shape=[
          jax.ShapeDtypeStruct((npad, 128), jnp.float32),
          jax.ShapeDtypeStruct((npad, 1), jnp.float32),
      ],
  )


def _tc_mid_body(p_ref, y_ref, dinv_ref, w_ref, b_ref, h_ref, y2_ref):
  dinv = dinv_ref[...]
  h = jnp.maximum((p_ref[0] + p_ref[1] + y_ref[...]) * dinv + b_ref[...], 0.0)
  h_ref[...] = h
  y2_ref[...] = jnp.dot(h, w_ref[...],
                        preferred_element_type=jnp.float32) * dinv


def _tc_mid(npad):
  return pl.pallas_call(
      _tc_mid_body,
      grid=(npad // BN,),
      in_specs=[
          pl.BlockSpec((2, BN, 128), lambda i: (0, i, 0)),
          pl.BlockSpec((BN, 128), lambda i: (i, 0)),
          pl.BlockSpec((BN, 1), lambda i: (i, 0)),
          pl.BlockSpec((128, 128), lambda i: (0, 0)),
          pl.BlockSpec((1, 128), lambda i: (0, 0)),
      ],
      out_specs=[
          pl.BlockSpec((BN, 128), lambda i: (i, 0)),
          pl.BlockSpec((BN, 128), lambda i: (i, 0)),
      ],
      out_shape=[
          jax.ShapeDtypeStruct((npad, 128), jnp.float32),
          jax.ShapeDtypeStruct((npad, 128), jnp.float32),
      ],
  )


def _tc_gatin_body(p_ref, y_ref, dinv_ref, wg_ref, b_ref, as_ref, ad_ref,
                   h_ref, xs_ref, asn_ref, adn_ref):
  dinv = dinv_ref[...]
  h = jnp.maximum((p_ref[0] + p_ref[1] + y_ref[...]) * dinv + b_ref[...], 0.0)
  xs = jnp.dot(h, wg_ref[...], preferred_element_type=jnp.float32)
  h_ref[...] = h
  xs_ref[...] = xs
  asn_ref[...] = jnp.dot(xs, as_ref[...], preferred_element_type=jnp.float32)
  adn_ref[...] = jnp.dot(xs, ad_ref[...], preferred_element_type=jnp.float32)


def _tc_gatin(npad):
  return pl.pallas_call(
      _tc_gatin_body,
      grid=(npad // BN,),
      in_specs=[
          pl.BlockSpec((2, BN, 128), lambda i: (0, i, 0)),
          pl.BlockSpec((BN, 128), lambda i: (i, 0)),
          pl.BlockSpec((BN, 1), lambda i: (i, 0)),
          pl.BlockSpec((128, 128), lambda i: (0, 0)),
          pl.BlockSpec((1, 128), lambda i: (0, 0)),
          pl.BlockSpec((128, 1), lambda i: (0, 0)),
          pl.BlockSpec((128, 1), lambda i: (0, 0)),
      ],
      out_specs=[
          pl.BlockSpec((BN, 128), lambda i: (i, 0)),
          pl.BlockSpec((BN, 128), lambda i: (i, 0)),
          pl.BlockSpec((BN, 1), lambda i: (i, 0)),
          pl.BlockSpec((BN, 1), lambda i: (i, 0)),
      ],
      out_shape=[
          jax.ShapeDtypeStruct((npad, 128), jnp.float32),
          jax.ShapeDtypeStruct((npad, 128), jnp.float32),
          jax.ShapeDtypeStruct((npad, 1), jnp.float32),
          jax.ShapeDtypeStruct((npad, 1), jnp.float32),
      ],
  )


def _tc_edge_body(ea_ref, we_ref, ate_ref, ae_ref, mx_ref, sm_ref):
  i = pl.program_id(0)
  web = jnp.dot(we_ref[...], ate_ref[...], preferred_element_type=jnp.float32)
  aeb = jnp.dot(ea_ref[...], web, preferred_element_type=jnp.float32)
  ae_ref[...] = aeb

  @pl.when(i == 0)
  def _():
    mx_ref[...] = jnp.full((1, 1), -jnp.inf, jnp.float32)
    sm_ref[...] = jnp.zeros((1, 16), jnp.float32)

  mx_ref[...] = jnp.maximum(mx_ref[...], jnp.max(aeb))
  sm_ref[...] = sm_ref[...] + jnp.sum(ea_ref[...], axis=0, keepdims=True)


def _tc_edge(epad2, eb):
  return pl.pallas_call(
      _tc_edge_body,
      grid=(epad2 // eb,),
      in_specs=[
          pl.BlockSpec((eb, 16), lambda i: (i, 0)),
          pl.BlockSpec((16, 128), lambda i: (0, 0)),
          pl.BlockSpec((128, 1), lambda i: (0, 0)),
      ],
      out_specs=[
          pl.BlockSpec((eb, 1), lambda i: (i, 0)),
          pl.BlockSpec((1, 1), lambda i: (0, 0)),
          pl.BlockSpec((1, 16), lambda i: (0, 0)),
      ],
      out_shape=[
          jax.ShapeDtypeStruct((epad2, 1), jnp.float32),
          jax.ShapeDtypeStruct((1, 1), jnp.float32),
          jax.ShapeDtypeStruct((1, 16), jnp.float32),
      ],
  )


def _make_tc_stats_body(e_real):
  def body(asn_ref, adn_ref, mx_ref, sm_ref, we_ref, ate_ref,
           avec_ref, exself_ref):
    web = jnp.dot(we_ref[...], ate_ref[...],
                  preferred_element_type=jnp.float32)
    aeloop = jnp.dot(sm_ref[...] / e_real, web,
                     preferred_element_type=jnp.float32)
    b = (jnp.max(asn_ref[...]) + jnp.max(adn_ref[...])
         + jnp.maximum(mx_ref[0, 0], aeloop[0, 0]))
    a = jnp.maximum(b, 0.2 * b)
    avec_ref[...] = jnp.full((1, 16), a, jnp.float32)
    al = asn_ref[...] + adn_ref[...] + aeloop
    al = jnp.maximum(al, 0.2 * al)
    exself_ref[...] = jnp.exp(al - a)
  return body


def _tc_stats(npad, e_real):
  return pl.pallas_call(
      _make_tc_stats_body(float(e_real)),
      grid=(1,),
      in_specs=[
          pl.BlockSpec((npad, 1), lambda i: (0, 0)),
          pl.BlockSpec((npad, 1), lambda i: (0, 0)),
          pl.BlockSpec((1, 1), lambda i: (0, 0)),
          pl.BlockSpec((1, 16), lambda i: (0, 0)),
          pl.BlockSpec((16, 128), lambda i: (0, 0)),
          pl.BlockSpec((128, 1), lambda i: (0, 0)),
      ],
      out_specs=[
          pl.BlockSpec((1, 16), lambda i: (0, 0)),
          pl.BlockSpec((npad, 1), lambda i: (0, 0)),
      ],
      out_shape=[
          jax.ShapeDtypeStruct((1, 16), jnp.float32),
          jax.ShapeDtypeStruct((npad, 1), jnp.float32),
      ],
  )


def _tc_final_body(np_ref, dp_ref, xs_ref, exs_ref, bg_ref, g_ref):
  exs = exs_ref[...]
  den = dp_ref[0, :, 0:1] + dp_ref[1, :, 0:1] + exs
  num = np_ref[0] + np_ref[1] + xs_ref[...] * exs
  g_ref[...] = num / den + bg_ref[...]


def _tc_final(npad):
  return pl.pallas_call(
      _tc_final_body,
      grid=(npad // BN,),
      in_specs=[
          pl.BlockSpec((2, BN, 128), lambda i: (0, i, 0)),
          pl.BlockSpec((2, BN, 16), lambda i: (0, i, 0)),
          pl.BlockSpec((BN, 128), lambda i: (i, 0)),
          pl.BlockSpec((BN, 1), lambda i: (i, 0)),
          pl.BlockSpec((1, 128), lambda i: (0, 0)),
      ],
      out_specs=pl.BlockSpec((BN, 128), lambda i: (i, 0)),
      out_shape=jax.ShapeDtypeStruct((npad, 128), jnp.float32),
  )


def kernel(x, edge_index, edge_attr, W1, b1, W2, b2, Wg, bg, We,
           att_src, att_dst, att_edge):
  n = x.shape[0]
  e = edge_index.shape[1]
  src = edge_index[0].astype(jnp.int32)
  dst = edge_index[1].astype(jnp.int32)

  npad = -(-(n + 1) // 128) * 128
  chunks = -(-(-(-e // (NW * CE))) // 4) * 4
  epad = NW * chunks * CE
  srcp = jnp.full((epad,), n, jnp.int32).at[:e].set(src).reshape(NW, chunks, CE)
  dstp = jnp.full((epad,), n, jnp.int32).at[:e].set(dst).reshape(NW, chunks, CE)
  z = jnp.zeros((npad, 128), jnp.float32)
  zd = jnp.zeros((npad, 16), jnp.float32)
  xpad = jnp.concatenate([x, jnp.zeros((npad - n, 128), jnp.float32)], 0)

  degp = _deg_call(npad, chunks)(zd, jnp.ones((CE, 16), jnp.float32), dstp)
  y1, dinv = _tc_pre(npad)(xpad, W1, degp)
  p1 = _seg_sum_call(npad, chunks)(y1, z, srcp, dstp)
  h1, y2 = _tc_mid(npad)(p1, y1, dinv, W2, b1.reshape(1, 128))
  p2 = _seg_sum_call(npad, chunks)(y2, z, srcp, dstp)
  h2, xs, asn, adn = _tc_gatin(npad)(
      p2, y2, dinv, Wg, b2.reshape(1, 128),
      att_src.reshape(128, 1), att_dst.reshape(128, 1))

  eb = 4000
  epad2 = -(-e // eb) * eb
  eap = jnp.concatenate(
      [edge_attr, jnp.zeros((epad2 - e, 16), jnp.float32)], 0)
  ae, aemx, easum = _tc_edge(epad2, eb)(eap, We, att_edge.reshape(128, 1))
  avec2, exself = _tc_stats(npad, e)(asn, adn, aemx, easum, We,
                                     att_edge.reshape(128, 1))

  asn16 = jnp.broadcast_to(asn, (npad, 16))
  adn16 = jnp.broadcast_to(adn, (npad, 16))
  aepf = jnp.zeros((epad,), jnp.float32).at[:e].set(ae[:e, 0])
  ae16 = jnp.broadcast_to(aepf[:, None], (epad, 16)).reshape(NW, chunks, CE, 16)
  avec = avec2.reshape(16)

  dparts = _gat_den_call(npad, chunks)(
      zd, srcp, dstp, ae16, asn16, adn16, avec)
  nparts = _gat_num_call(npad, chunks)(
      xs, z, srcp, dstp, ae16, asn16, adn16, avec)
  g = _tc_final(npad)(nparts, dparts, xs, exself, bg.reshape(1, 128))

  return jnp.stack([h1[:n], h2[:n], g[:n]], axis=0)


# speedup vs baseline: 1.0842x; 1.0842x over previous
import functools

import jax
import jax.numpy as jnp
from jax import lax
from jax.experimental import pallas as pl
from jax.experimental.pallas import tpu as pltpu
from jax.experimental.pallas import tpu_sc as plsc

NC = 2
NS = 16
NW = NC * NS
CE = 128
L = 16


def _mesh():
  return plsc.VectorSubcoreMesh(core_axis_name="c", subcore_axis_name="s")


def _deg_call(npad, chunks):
  rps = npad // NS

  @functools.partial(
      pl.kernel,
      out_type=jax.ShapeDtypeStruct((NC, npad, 16), jnp.float32),
      mesh=_mesh(),
      compiler_params=pltpu.CompilerParams(use_tc_tiling_on_sc=False),
      scratch_types=[
          pltpu.VMEM((chunks, CE), jnp.int32),
          pltpu.VMEM((CE, 16), jnp.float32),
          pltpu.VMEM_SHARED((npad, 16), jnp.float32),
      ],
  )
  def k(zd_hbm, one_hbm, dst_hbm, out_hbm, didx, ones, accd):
    c = lax.axis_index("c")
    s = lax.axis_index("s")
    wid = s * NC + c
    pltpu.sync_copy(zd_hbm.at[pl.ds(s * rps, rps)], accd.at[pl.ds(s * rps, rps)])
    pltpu.sync_copy(dst_hbm.at[wid], didx)
    pltpu.sync_copy(one_hbm, ones)
    plsc.subcore_barrier()

    def body(i, carry):
      pltpu.sync_copy(ones, accd.at[didx.at[i]], add=True)
      return carry

    lax.fori_loop(0, chunks, body, 0)
    plsc.subcore_barrier()
    pltpu.sync_copy(accd.at[pl.ds(s * rps, rps)],
                    out_hbm.at[c, pl.ds(s * rps, rps)])

  return k


def _seg_sum_call(npad, chunks):
  rps = npad // NS

  @functools.partial(
      pl.kernel,
      out_type=jax.ShapeDtypeStruct((NC, npad, 128), jnp.float32),
      mesh=_mesh(),
      scratch_types=[
          pltpu.VMEM((chunks, CE), jnp.int32),
          pltpu.VMEM((chunks, CE), jnp.int32),
          pltpu.VMEM((CE, 128), jnp.float32),
          pltpu.VMEM_SHARED((npad, 128), jnp.float32),
          pltpu.SemaphoreType.DMA,
      ],
  )
  def k(y_hbm, z_hbm, src_hbm, dst_hbm, out_hbm, sidx, didx, rows, acc, sem):
    c = lax.axis_index("c")
    s = lax.axis_index("s")
    wid = s * NC + c
    pltpu.sync_copy(z_hbm.at[pl.ds(s * rps, rps)], acc.at[pl.ds(s * rps, rps)])
    pltpu.sync_copy(src_hbm.at[wid], sidx)
    pltpu.sync_copy(dst_hbm.at[wid], didx)
    plsc.subcore_barrier()

    def body(i, carry):
      pltpu.async_copy(y_hbm.at[sidx.at[i]], rows, sem).wait()
      pltpu.sync_copy(rows, acc.at[didx.at[i]], add=True)
      return carry

    lax.fori_loop(0, chunks, body, 0)
    plsc.subcore_barrier()
    pltpu.sync_copy(acc.at[pl.ds(s * rps, rps)],
                    out_hbm.at[c, pl.ds(s * rps, rps)])

  return k


def _gat_den_call(npad, chunks):
  rps = npad // NS

  @functools.partial(
      pl.kernel,
      out_type=jax.ShapeDtypeStruct((NC, npad, 16), jnp.float32),
      mesh=_mesh(),
      compiler_params=pltpu.CompilerParams(use_tc_tiling_on_sc=False),
      scratch_types=[
          pltpu.VMEM((chunks, CE), jnp.int32),
          pltpu.VMEM((chunks, CE), jnp.int32),
          pltpu.VMEM((16,), jnp.float32),
          pltpu.VMEM((CE, 16), jnp.float32),
          pltpu.VMEM((CE, 16), jnp.float32),
          pltpu.VMEM((CE, 16), jnp.float32),
          pltpu.VMEM((CE, 16), jnp.float32),
          pltpu.VMEM_SHARED((npad, 16), jnp.float32),
          pltpu.SemaphoreType.DMA,
      ],
  )
  def k(zd_hbm, src_hbm, dst_hbm, ae_hbm, asn_hbm, adn_hbm, avec_hbm,
        outd_hbm, sidx, didx, avecv, asr, adr, aer, exrows, accd, sem):
    c = lax.axis_index("c")
    s = lax.axis_index("s")
    wid = s * NC + c
    pltpu.sync_copy(zd_hbm.at[pl.ds(s * rps, rps)], accd.at[pl.ds(s * rps, rps)])
    pltpu.sync_copy(src_hbm.at[wid], sidx)
    pltpu.sync_copy(dst_hbm.at[wid], didx)
    pltpu.sync_copy(avec_hbm, avecv)
    plsc.subcore_barrier()
    A = avecv[...]

    def chunk_body(i, carry):
      c2 = pltpu.async_copy(asn_hbm.at[sidx.at[i]], asr, sem)
      c3 = pltpu.async_copy(adn_hbm.at[didx.at[i]], adr, sem)
      pltpu.sync_copy(ae_hbm.at[wid, i], aer)
      c2.wait()
      c3.wait()

      def edge_body(j, cc):
        al = asr[j, :] + adr[j, :] + aer[j, :]
        al = jnp.maximum(al, 0.2 * al)
        exrows[j, :] = jnp.exp(al - A)
        return cc

      lax.fori_loop(0, CE, edge_body, 0)
      pltpu.sync_copy(exrows, accd.at[didx.at[i]], add=True)
      return carry

    lax.fori_loop(0, chunks, chunk_body, 0)
    plsc.subcore_barrier()
    pltpu.sync_copy(accd.at[pl.ds(s * rps, rps)],
                    outd_hbm.at[c, pl.ds(s * rps, rps)])

  return k


def _gat_num_call(npad, chunks):
  rps = npad // NS

  @functools.partial(
      pl.kernel,
      out_type=jax.ShapeDtypeStruct((NC, npad, 128), jnp.float32),
      mesh=_mesh(),
      compiler_params=pltpu.CompilerParams(use_tc_tiling_on_sc=False),
      scratch_types=[
          pltpu.VMEM((chunks, CE), jnp.int32),
          pltpu.VMEM((chunks, CE), jnp.int32),
          pltpu.VMEM((16,), jnp.float32),
          pltpu.VMEM((CE, 128), jnp.float32),
          pltpu.VMEM((CE, 16), jnp.float32),
          pltpu.VMEM((CE, 16), jnp.float32),
          pltpu.VMEM((CE, 16), jnp.float32),
          pltpu.VMEM_SHARED((npad, 128), jnp.float32),
          pltpu.SemaphoreType.DMA,
      ],
  )
  def k(xs_hbm, zn_hbm, src_hbm, dst_hbm, ae_hbm, asn_hbm, adn_hbm,
        avec_hbm, outn_hbm, sidx, didx, avecv, rows, asr, adr, aer, accn,
        sem):
    c = lax.axis_index("c")
    s = lax.axis_index("s")
    wid = s * NC + c
    pltpu.sync_copy(zn_hbm.at[pl.ds(s * rps, rps)], accn.at[pl.ds(s * rps, rps)])
    pltpu.sync_copy(src_hbm.at[wid], sidx)
    pltpu.sync_copy(dst_hbm.at[wid], didx)
    pltpu.sync_copy(avec_hbm, avecv)
    plsc.subcore_barrier()
    A = avecv[...]

    def chunk_body(i, carry):
      c1 = pltpu.async_copy(xs_hbm.at[sidx.at[i]], rows, sem)
      c2 = pltpu.async_copy(asn_hbm.at[sidx.at[i]], asr, sem)
      c3 = pltpu.async_copy(adn_hbm.at[didx.at[i]], adr, sem)
      pltpu.sync_copy(ae_hbm.at[wid, i], aer)
      c1.wait()
      c2.wait()
      c3.wait()

      def edge_body(j, cc):
        al = asr[j, :] + adr[j, :] + aer[j, :]
        al = jnp.maximum(al, 0.2 * al)
        exr = jnp.exp(al - A)
        for r in range(128 // L):
          rsl = pl.ds(r * L, L)
          rows[j, rsl] = rows[j, rsl] * exr
        return cc

      lax.fori_loop(0, CE, edge_body, 0)
      pltpu.sync_copy(rows, accn.at[didx.at[i]], add=True)
      return carry

    lax.fori_loop(0, chunks, chunk_body, 0)
    plsc.subcore_barrier()
    pltpu.sync_copy(accn.at[pl.ds(s * rps, rps)],
                    outn_hbm.at[c, pl.ds(s * rps, rps)])

  return k


BN = 128


def _tc_pre_body(x_ref, w_ref, degp_ref, y_ref, dinv_ref):
  deg = degp_ref[0, :, 0:1] + degp_ref[1, :, 0:1] + 1.0
  dinv = lax.rsqrt(jnp.maximum(deg, 1.0))
  y_ref[...] = jnp.dot(x_ref[...], w_ref[...],
                       preferred_element_type=jnp.float32) * dinv
  dinv_ref[...] = dinv


def _tc_pre(npad):
  return pl.pallas_call(
      _tc_pre_body,
      grid=(npad // BN,),
      in_specs=[
          pl.BlockSpec((BN, 128), lambda i: (i, 0)),
          pl.BlockSpec((128, 128), lambda i: (0, 0)),
          pl.BlockSpec((2, BN, 16), lambda i: (0, i, 0)),
      ],
      out_specs=[
          pl.BlockSpec((BN, 128), lambda i: (i, 0)),
          pl.BlockSpec((BN, 1), lambda i: (i, 0)),
      ],
      out_shape=[
          jax.ShapeDtypeStruct((npad, 128), jnp.float32),
          jax.ShapeDtypeStruct((npad, 1), jnp.float32),
      ],
  )


def _tc_mid_body(p_ref, y_ref, dinv_ref, w_ref, b_ref, h_ref, y2_ref):
  dinv = dinv_ref[...]
  h = jnp.maximum((p_ref[0] + p_ref[1] + y_ref[...]) * dinv + b_ref[...], 0.0)
  h_ref[...] = h
  y2_ref[...] = jnp.dot(h, w_ref[...],
                        preferred_element_type=jnp.float32) * dinv


def _tc_mid(npad):
  return pl.pallas_call(
      _tc_mid_body,
      grid=(npad // BN,),
      in_specs=[
          pl.BlockSpec((2, BN, 128), lambda i: (0, i, 0)),
          pl.BlockSpec((BN, 128), lambda i: (i, 0)),
          pl.BlockSpec((BN, 1), lambda i: (i, 0)),
          pl.BlockSpec((128, 128), lambda i: (0, 0)),
          pl.BlockSpec((1, 128), lambda i: (0, 0)),
      ],
      out_specs=[
          pl.BlockSpec((BN, 128), lambda i: (i, 0)),
          pl.BlockSpec((BN, 128), lambda i: (i, 0)),
      ],
      out_shape=[
          jax.ShapeDtypeStruct((npad, 128), jnp.float32),
          jax.ShapeDtypeStruct((npad, 128), jnp.float32),
      ],
  )


def _tc_gatin_body(p_ref, y_ref, dinv_ref, wg_ref, b_ref, as_ref, ad_ref,
                   h_ref, xs_ref, asn_ref, adn_ref):
  dinv = dinv_ref[...]
  h = jnp.maximum((p_ref[0] + p_ref[1] + y_ref[...]) * dinv + b_ref[...], 0.0)
  xs = jnp.dot(h, wg_ref[...], preferred_element_type=jnp.float32)
  h_ref[...] = h
  xs_ref[...] = xs
  asn_ref[...] = jnp.dot(xs, as_ref[...], preferred_element_type=jnp.float32)
  adn_ref[...] = jnp.dot(xs, ad_ref[...], preferred_element_type=jnp.float32)


def _tc_gatin(npad):
  return pl.pallas_call(
      _tc_gatin_body,
      grid=(npad // BN,),
      in_specs=[
          pl.BlockSpec((2, BN, 128), lambda i: (0, i, 0)),
          pl.BlockSpec((BN, 128), lambda i: (i, 0)),
          pl.BlockSpec((BN, 1), lambda i: (i, 0)),
          pl.BlockSpec((128, 128), lambda i: (0, 0)),
          pl.BlockSpec((1, 128), lambda i: (0, 0)),
          pl.BlockSpec((128, 1), lambda i: (0, 0)),
          pl.BlockSpec((128, 1), lambda i: (0, 0)),
      ],
      out_specs=[
          pl.BlockSpec((BN, 128), lambda i: (i, 0)),
          pl.BlockSpec((BN, 128), lambda i: (i, 0)),
          pl.BlockSpec((BN, 1), lambda i: (i, 0)),
          pl.BlockSpec((BN, 1), lambda i: (i, 0)),
      ],
      out_shape=[
          jax.ShapeDtypeStruct((npad, 128), jnp.float32),
          jax.ShapeDtypeStruct((npad, 128), jnp.float32),
          jax.ShapeDtypeStruct((npad, 1), jnp.float32),
          jax.ShapeDtypeStruct((npad, 1), jnp.float32),
      ],
  )


def _tc_edge_body(ea_ref, we_ref, ate_ref, ae_ref, mx_ref, sm_ref):
  i = pl.program_id(0)
  web = jnp.dot(we_ref[...], ate_ref[...], preferred_element_type=jnp.float32)
  aeb = jnp.dot(ea_ref[...], web, preferred_element_type=jnp.float32)
  ae_ref[...] = aeb

  @pl.when(i == 0)
  def _():
    mx_ref[...] = jnp.full((1, 1), -jnp.inf, jnp.float32)
    sm_ref[...] = jnp.zeros((1, 16), jnp.float32)

  mx_ref[...] = jnp.maximum(mx_ref[...], jnp.max(aeb))
  sm_ref[...] = sm_ref[...] + jnp.sum(ea_ref[...], axis=0, keepdims=True)


def _tc_edge(epad2, eb):
  return pl.pallas_call(
      _tc_edge_body,
      grid=(epad2 // eb,),
      in_specs=[
          pl.BlockSpec((eb, 16), lambda i: (i, 0)),
          pl.BlockSpec((16, 128), lambda i: (0, 0)),
          pl.BlockSpec((128, 1), lambda i: (0, 0)),
      ],
      out_specs=[
          pl.BlockSpec((eb, 1), lambda i: (i, 0)),
          pl.BlockSpec((1, 1), lambda i: (0, 0)),
          pl.BlockSpec((1, 16), lambda i: (0, 0)),
      ],
      out_shape=[
          jax.ShapeDtypeStruct((epad2, 1), jnp.float32),
          jax.ShapeDtypeStruct((1, 1), jnp.float32),
          jax.ShapeDtypeStruct((1, 16), jnp.float32),
      ],
  )


def _make_tc_stats_body(e_real):
  def body(asn_ref, adn_ref, mx_ref, sm_ref, we_ref, ate_ref,
           avec_ref, exself_ref):
    web = jnp.dot(we_ref[...], ate_ref[...],
                  preferred_element_type=jnp.float32)
    aeloop = jnp.dot(sm_ref[...] / e_real, web,
                     preferred_element_type=jnp.float32)
    b = (jnp.max(asn_ref[...]) + jnp.max(adn_ref[...])
         + jnp.maximum(mx_ref[0, 0], aeloop[0, 0]))
    a = jnp.maximum(b, 0.2 * b)
    avec_ref[...] = jnp.full((1, 16), a, jnp.float32)
    al = asn_ref[...] + adn_ref[...] + aeloop
    al = jnp.maximum(al, 0.2 * al)
    exself_ref[...] = jnp.exp(al - a)
  return body


def _tc_stats(npad, e_real):
  return pl.pallas_call(
      _make_tc_stats_body(float(e_real)),
      grid=(1,),
      in_specs=[
          pl.BlockSpec((npad, 1), lambda i: (0, 0)),
          pl.BlockSpec((npad, 1), lambda i: (0, 0)),
          pl.BlockSpec((1, 1), lambda i: (0, 0)),
          pl.BlockSpec((1, 16), lambda i: (0, 0)),
          pl.BlockSpec((16, 128), lambda i: (0, 0)),
          pl.BlockSpec((128, 1), lambda i: (0, 0)),
      ],
      out_specs=[
          pl.BlockSpec((1, 16), lambda i: (0, 0)),
          pl.BlockSpec((npad, 1), lambda i: (0, 0)),
      ],
      out_shape=[
          jax.ShapeDtypeStruct((1, 16), jnp.float32),
          jax.ShapeDtypeStruct((npad, 1), jnp.float32),
      ],
  )


def _tc_final_body(np_ref, dp_ref, xs_ref, exs_ref, bg_ref, g_ref):
  exs = exs_ref[...]
  den = dp_ref[0, :, 0:1] + dp_ref[1, :, 0:1] + exs
  num = np_ref[0] + np_ref[1] + xs_ref[...] * exs
  g_ref[...] = num / den + bg_ref[...]


def _tc_final(npad):
  return pl.pallas_call(
      _tc_final_body,
      grid=(npad // BN,),
      in_specs=[
          pl.BlockSpec((2, BN, 128), lambda i: (0, i, 0)),
          pl.BlockSpec((2, BN, 16), lambda i: (0, i, 0)),
          pl.BlockSpec((BN, 128), lambda i: (i, 0)),
          pl.BlockSpec((BN, 1), lambda i: (i, 0)),
          pl.BlockSpec((1, 128), lambda i: (0, 0)),
      ],
      out_specs=pl.BlockSpec((BN, 128), lambda i: (i, 0)),
      out_shape=jax.ShapeDtypeStruct((npad, 128), jnp.float32),
  )


def kernel(x, edge_index, edge_attr, W1, b1, W2, b2, Wg, bg, We,
           att_src, att_dst, att_edge):
  n = x.shape[0]
  e = edge_index.shape[1]
  src = edge_index[0].astype(jnp.int32)
  dst = edge_index[1].astype(jnp.int32)

  npad = -(-(n + 1) // 128) * 128
  chunks = -(-e // (NW * CE))
  epad = NW * chunks * CE
  srcp = jnp.full((epad,), n, jnp.int32).at[:e].set(src).reshape(NW, chunks, CE)
  dstp = jnp.full((epad,), n, jnp.int32).at[:e].set(dst).reshape(NW, chunks, CE)
  z = jnp.zeros((npad, 128), jnp.float32)
  zd = jnp.zeros((npad, 16), jnp.float32)
  xpad = jnp.concatenate([x, jnp.zeros((npad - n, 128), jnp.float32)], 0)

  degp = _deg_call(npad, chunks)(zd, jnp.ones((CE, 16), jnp.float32), dstp)
  y1, dinv = _tc_pre(npad)(xpad, W1, degp)
  p1 = _seg_sum_call(npad, chunks)(y1, z, srcp, dstp)
  h1, y2 = _tc_mid(npad)(p1, y1, dinv, W2, b1.reshape(1, 128))
  p2 = _seg_sum_call(npad, chunks)(y2, z, srcp, dstp)
  h2, xs, asn, adn = _tc_gatin(npad)(
      p2, y2, dinv, Wg, b2.reshape(1, 128),
      att_src.reshape(128, 1), att_dst.reshape(128, 1))

  eb = 4000
  epad2 = -(-e // eb) * eb
  eap = jnp.concatenate(
      [edge_attr, jnp.zeros((epad2 - e, 16), jnp.float32)], 0)
  ae, aemx, easum = _tc_edge(epad2, eb)(eap, We, att_edge.reshape(128, 1))
  avec2, exself = _tc_stats(npad, e)(asn, adn, aemx, easum, We,
                                     att_edge.reshape(128, 1))

  asn16 = jnp.broadcast_to(asn, (npad, 16))
  adn16 = jnp.broadcast_to(adn, (npad, 16))
  aepf = jnp.zeros((epad,), jnp.float32).at[:e].set(ae[:e, 0])
  ae16 = jnp.broadcast_to(aepf[:, None], (epad, 16)).reshape(NW, chunks, CE, 16)
  avec = avec2.reshape(16)

  dparts = _gat_den_call(npad, chunks)(
      zd, srcp, dstp, ae16, asn16, adn16, avec)
  nparts = _gat_num_call(npad, chunks)(
      xs, z, srcp, dstp, ae16, asn16, adn16, avec)
  g = _tc_final(npad)(nparts, dparts, xs, exself, bg.reshape(1, 128))

  return jnp.stack([h1[:n], h2[:n], g[:n]], axis=0)
